# baseline (speedup 1.0000x reference)
import functools

import jax
import jax.numpy as jnp
from jax import lax
from jax.experimental import pallas as pl
from jax.experimental.pallas import tpu as pltpu
from jax.experimental.pallas import tpu_sc as plsc

B = 16384
RANK = 16
NROWS = 1000000

DW = 65536
NCH = 16
FSTRIDE = DW * NCH
FPAIRS = FSTRIDE // 2
NPAIRS = NROWS // 2 + 1

_info = plsc.get_sparse_core_info()
NC = _info.num_cores
NS = _info.num_subcores
L = _info.num_lanes
NW = NC * NS
BPW = B // NW
BLKS = BPW // L

_mesh = plsc.VectorSubcoreMesh(core_axis_name="c", subcore_axis_name="s")

_SC_PARAMS = pltpu.CompilerParams(
    needs_layout_passes=False, use_tc_tiling_on_sc=False)


def _detile_body(t_ref, flat_ref, bf_ref, sem):
    c = pl.program_id(0)
    for f in range(RANK):
        bf_ref[pl.ds(f * DW, DW)] = t_ref[f].astype(jnp.bfloat16)
    cps = []
    for f in range(RANK):
        cps.append(pltpu.make_async_copy(
            bf_ref.at[pl.ds(f * DW, DW)],
            flat_ref.at[pl.ds(f * FSTRIDE + c * DW, DW)],
            sem,
        ))
    for cp in cps:
        cp.start()
    for cp in cps:
        cp.wait()


def _detile(t):
    return pl.pallas_call(
        _detile_body,
        grid=(NCH,),
        in_specs=[pl.BlockSpec((RANK, DW), lambda c: (0, c))],
        out_specs=pl.BlockSpec(memory_space=pl.ANY),
        out_shape=jax.ShapeDtypeStruct((RANK * FSTRIDE,), jnp.bfloat16),
        scratch_shapes=[pltpu.VMEM((RANK * DW,), jnp.bfloat16),
                        pltpu.SemaphoreType.DMA],
    )(t)


def _gather_pairs(rows, pidx_v, buf_v, sem):
    def blk(b, carry):
        pids = pidx_v[pl.ds(b * L, L)] >> 1
        d = pl.ds(b * L, L)
        for f in range(RANK):
            pltpu.async_copy(rows[f].at[pids], buf_v.at[f, d], sem)
        return carry

    lax.fori_loop(0, BLKS, blk, 0)


def _select_half(v, odd):
    lo = v << 16
    hi = v & jnp.int32(-65536)
    return plsc.bitcast(jnp.where(odd, hi, lo), jnp.float32)


@functools.partial(
    pl.kernel,
    mesh=_mesh,
    out_type=(
        jax.ShapeDtypeStruct((RANK * B,), jnp.float32),
        jax.ShapeDtypeStruct((RANK * B,), jnp.float32),
    ),
    scratch_types=[
        pltpu.VMEM((BPW,), jnp.int32),
        pltpu.VMEM((BPW,), jnp.int32),
        pltpu.VMEM((RANK, BPW), jnp.int32),
        pltpu.VMEM((RANK, BPW), jnp.int32),
        pltpu.VMEM((RANK, BPW), jnp.float32),
        pltpu.VMEM((RANK, BPW), jnp.float32),
        pltpu.SemaphoreType.DMA,
    ],
    compiler_params=_SC_PARAMS,
)
def _sc_items(it_hbm, pids_hbm, nids_hbm, pfeat_hbm, nfeat_hbm,
              pidx_v, nidx_v, praw_v, nraw_v, pbuf_v, nbuf_v, sem):
    wid = lax.axis_index("s") * NC + lax.axis_index("c")
    sl = pl.ds(wid * BPW, BPW)

    pltpu.sync_copy(pids_hbm.at[sl], pidx_v)
    pltpu.sync_copy(nids_hbm.at[sl], nidx_v)

    irows = [it_hbm.at[pl.ds(f * FPAIRS, NPAIRS)] for f in range(RANK)]
    _gather_pairs(irows, pidx_v, praw_v, sem)
    _gather_pairs(irows, nidx_v, nraw_v, sem)

    proto = it_hbm.at[pl.ds(0, RANK * BPW)]
    pltpu.make_async_copy(proto, praw_v, sem).wait()
    pltpu.make_async_copy(proto, nraw_v, sem).wait()

    def sel_body(b, carry):
        d = pl.ds(b * L, L)
        podd = (pidx_v[d] & 1) == 1
        nodd = (nidx_v[d] & 1) == 1
        for f in range(RANK):
            pbuf_v[f, d] = _select_half(praw_v[f, d], podd)
            nbuf_v[f, d] = _select_half(nraw_v[f, d], nodd)
        return carry

    lax.fori_loop(0, BLKS, sel_body, 0)

    for f in range(RANK):
        out = pl.ds(f * B + wid * BPW, BPW)
        pltpu.sync_copy(pbuf_v.at[f], pfeat_hbm.at[out])
        pltpu.sync_copy(nbuf_v.at[f], nfeat_hbm.at[out])


@functools.partial(
    pl.kernel,
    mesh=_mesh,
    out_type=(
        jax.ShapeDtypeStruct((B,), jnp.float32),
        jax.ShapeDtypeStruct((B,), jnp.float32),
    ),
    scratch_types=[
        pltpu.VMEM((BPW,), jnp.int32),
        pltpu.VMEM((RANK, BPW), jnp.int32),
        pltpu.VMEM((RANK, BPW), jnp.float32),
        pltpu.VMEM((RANK, BPW), jnp.float32),
        pltpu.VMEM((BPW,), jnp.float32),
        pltpu.VMEM((BPW,), jnp.float32),
        pltpu.SemaphoreType.DMA,
        pltpu.SemaphoreType.DMA,
    ],
    compiler_params=_SC_PARAMS,
)
def _sc_user_score(ut_hbm, uids_hbm, pfeat_hbm, nfeat_hbm,
                   outp_hbm, outn_hbm,
                   uidx_v, uraw_v, pbuf_v, nbuf_v,
                   outp_v, outn_v, semG, semS):
    wid = lax.axis_index("s") * NC + lax.axis_index("c")
    sl = pl.ds(wid * BPW, BPW)

    pltpu.sync_copy(uids_hbm.at[sl], uidx_v)

    urows = [ut_hbm.at[pl.ds(f * FPAIRS, NPAIRS)] for f in range(RANK)]

    def blk(b, carry):
        pids = (uidx_v[pl.ds(b * L, L)] >> 1)
        d = pl.ds(b * L, L)
        for f in range(RANK):
            pltpu.async_copy(urows[f].at[pids], uraw_v.at[f, d], semG)
        return carry

    lax.fori_loop(0, BLKS, blk, 0)

    stage = []
    for f in range(RANK):
        src = pl.ds(f * B + wid * BPW, BPW)
        stage.append(pltpu.async_copy(pfeat_hbm.at[src], pbuf_v.at[f], semS))
        stage.append(pltpu.async_copy(nfeat_hbm.at[src], nbuf_v.at[f], semS))
    for c in stage:
        c.wait()
    proto = pfeat_hbm.at[pl.ds(0, RANK * BPW)]
    pltpu.make_async_copy(proto, uraw_v, semG).wait()

    def blk_body(b, carry):
        base = b * L
        d = pl.ds(base, L)
        uodd = (uidx_v[d] & 1) == 1
        accp = jnp.zeros((L,), jnp.float32)
        accn = jnp.zeros((L,), jnp.float32)
        for f in range(RANK):
            u = _select_half(uraw_v[f, d], uodd)
            accp = accp + u * pbuf_v[f, d]
            accn = accn + u * nbuf_v[f, d]
        outp_v[d] = accp
        outn_v[d] = accn
        return carry

    lax.fori_loop(0, BLKS, blk_body, 0)

    pltpu.sync_copy(outp_v, outp_hbm.at[sl])
    pltpu.sync_copy(outn_v, outn_hbm.at[sl])


def _as_pairs(flat_bf16):
    return jax.lax.bitcast_convert_type(
        flat_bf16.reshape(-1, 2), jnp.int32)


def kernel(user_ids, pos_items, neg_items, user_emb, item_emb):
    iflat = _as_pairs(_detile(item_emb.T))
    pfeat, nfeat = _sc_items(iflat,
                             pos_items.astype(jnp.int32),
                             neg_items.astype(jnp.int32))
    uflat = _as_pairs(_detile(user_emb.T))
    return _sc_user_score(uflat, user_ids.astype(jnp.int32), pfeat, nfeat)


# speedup vs baseline: 93.1591x; 93.1591x over previous
import functools

import jax
import jax.numpy as jnp
from jax import lax
from jax.experimental import pallas as pl
from jax.experimental.pallas import tpu as pltpu
from jax.experimental.pallas import tpu_sc as plsc

B = 16384
RANK = 16
NROWS = 1000000

DW = 65536
NCH = 16
FSTRIDE = DW * NCH

_info = plsc.get_sparse_core_info()
NC = _info.num_cores
NS = _info.num_subcores
L = _info.num_lanes
NW = NC * NS
BPW = B // NW
BLKS = BPW // L

_mesh = plsc.VectorSubcoreMesh(core_axis_name="c", subcore_axis_name="s")

_SC_PARAMS = pltpu.CompilerParams(
    needs_layout_passes=False, use_tc_tiling_on_sc=False)


def _detile_body(t_ref, flat_ref, sem):
    c = pl.program_id(0)
    cps = []
    for f in range(RANK):
        cps.append(pltpu.make_async_copy(
            t_ref.at[f],
            flat_ref.at[pl.ds(f * FSTRIDE + c * DW, DW)],
            sem,
        ))
    for cp in cps:
        cp.start()
    for cp in cps:
        cp.wait()


def _detile(t):
    return pl.pallas_call(
        _detile_body,
        grid=(NCH,),
        in_specs=[pl.BlockSpec((RANK, DW), lambda c: (0, c))],
        out_specs=pl.BlockSpec(memory_space=pl.ANY),
        out_shape=jax.ShapeDtypeStruct((RANK * FSTRIDE,), jnp.float32),
        scratch_shapes=[pltpu.SemaphoreType.DMA],
    )(t)


def _gather_feats(rows, idx_v, buf_v, sem):
    def blk(b, carry):
        ids = idx_v[pl.ds(b * L, L)]
        d = pl.ds(b * L, L)
        for f in range(RANK):
            pltpu.async_copy(rows[f].at[ids], buf_v.at[f, d], sem)
        return carry

    lax.fori_loop(0, BLKS, blk, 0)


@functools.partial(
    pl.kernel,
    mesh=_mesh,
    out_type=(
        jax.ShapeDtypeStruct((RANK * B,), jnp.float32),
        jax.ShapeDtypeStruct((RANK * B,), jnp.float32),
    ),
    scratch_types=[
        pltpu.VMEM((BPW,), jnp.int32),
        pltpu.VMEM((BPW,), jnp.int32),
        pltpu.VMEM((RANK, BPW), jnp.float32),
        pltpu.VMEM((RANK, BPW), jnp.float32),
        pltpu.SemaphoreType.DMA,
    ],
    compiler_params=_SC_PARAMS,
)
def _sc_items(it_hbm, pids_hbm, nids_hbm, pfeat_hbm, nfeat_hbm,
              pidx_v, nidx_v, pbuf_v, nbuf_v, sem):
    wid = lax.axis_index("s") * NC + lax.axis_index("c")
    sl = pl.ds(wid * BPW, BPW)

    pltpu.sync_copy(pids_hbm.at[sl], pidx_v)
    pltpu.sync_copy(nids_hbm.at[sl], nidx_v)

    irows = [it_hbm.at[pl.ds(f * FSTRIDE, NROWS)] for f in range(RANK)]
    _gather_feats(irows, pidx_v, pbuf_v, sem)
    _gather_feats(irows, nidx_v, nbuf_v, sem)

    proto = it_hbm.at[pl.ds(0, RANK * BPW)]
    pltpu.make_async_copy(proto, pbuf_v, sem).wait()
    pltpu.make_async_copy(proto, nbuf_v, sem).wait()

    for f in range(RANK):
        out = pl.ds(f * B + wid * BPW, BPW)
        pltpu.sync_copy(pbuf_v.at[f], pfeat_hbm.at[out])
        pltpu.sync_copy(nbuf_v.at[f], nfeat_hbm.at[out])


@functools.partial(
    pl.kernel,
    mesh=_mesh,
    out_type=(
        jax.ShapeDtypeStruct((B,), jnp.float32),
        jax.ShapeDtypeStruct((B,), jnp.float32),
    ),
    scratch_types=[
        pltpu.VMEM((BPW,), jnp.int32),
        pltpu.VMEM((RANK, BPW), jnp.float32),
        pltpu.VMEM((RANK, BPW), jnp.float32),
        pltpu.VMEM((RANK, BPW), jnp.float32),
        pltpu.VMEM((BPW,), jnp.float32),
        pltpu.VMEM((BPW,), jnp.float32),
        pltpu.SemaphoreType.DMA,
        pltpu.SemaphoreType.DMA,
    ],
    compiler_params=_SC_PARAMS,
)
def _sc_user_score(ut_hbm, uids_hbm, pfeat_hbm, nfeat_hbm,
                   outp_hbm, outn_hbm,
                   uidx_v, ubuf_v, pbuf_v, nbuf_v,
                   outp_v, outn_v, semG, semS):
    wid = lax.axis_index("s") * NC + lax.axis_index("c")
    sl = pl.ds(wid * BPW, BPW)

    pltpu.sync_copy(uids_hbm.at[sl], uidx_v)

    urows = [ut_hbm.at[pl.ds(f * FSTRIDE, NROWS)] for f in range(RANK)]
    _gather_feats(urows, uidx_v, ubuf_v, semG)

    stage = []
    for f in range(RANK):
        src = pl.ds(f * B + wid * BPW, BPW)
        stage.append(pltpu.async_copy(pfeat_hbm.at[src], pbuf_v.at[f], semS))
        stage.append(pltpu.async_copy(nfeat_hbm.at[src], nbuf_v.at[f], semS))
    for c in stage:
        c.wait()
    proto = ut_hbm.at[pl.ds(0, RANK * BPW)]
    pltpu.make_async_copy(proto, ubuf_v, semG).wait()

    def blk_body(b, carry):
        base = b * L
        accp = jnp.zeros((L,), jnp.float32)
        accn = jnp.zeros((L,), jnp.float32)
        for f in range(RANK):
            u = ubuf_v[f, pl.ds(base, L)]
            p = pbuf_v[f, pl.ds(base, L)]
            n = nbuf_v[f, pl.ds(base, L)]
            accp = accp + u * p
            accn = accn + u * n
        outp_v[pl.ds(base, L)] = accp
        outn_v[pl.ds(base, L)] = accn
        return carry

    lax.fori_loop(0, BLKS, blk_body, 0)

    pltpu.sync_copy(outp_v, outp_hbm.at[sl])
    pltpu.sync_copy(outn_v, outn_hbm.at[sl])


def kernel(user_ids, pos_items, neg_items, user_emb, item_emb):
    iflat = _detile(item_emb.T)
    pfeat, nfeat = _sc_items(iflat,
                             pos_items.astype(jnp.int32),
                             neg_items.astype(jnp.int32))
    uflat = _detile(user_emb.T)
    return _sc_user_score(uflat, user_ids.astype(jnp.int32), pfeat, nfeat)


# speedup vs baseline: 94.0042x; 1.0091x over previous
import functools

import jax
import jax.numpy as jnp
from jax import lax
from jax.experimental import pallas as pl
from jax.experimental.pallas import tpu as pltpu
from jax.experimental.pallas import tpu_sc as plsc

B = 16384
RANK = 16
NROWS = 1000000

DW = 131072
NCH = 8
FSTRIDE = DW * NCH

_info = plsc.get_sparse_core_info()
NC = _info.num_cores
NS = _info.num_subcores
L = _info.num_lanes
NW = NC * NS
BPW = B // NW
BLKS = BPW // L

_mesh = plsc.VectorSubcoreMesh(core_axis_name="c", subcore_axis_name="s")

_SC_PARAMS = pltpu.CompilerParams(
    needs_layout_passes=False, use_tc_tiling_on_sc=False)


def _detile_body(t_ref, flat_ref, sem):
    c = pl.program_id(0)
    cps = []
    for f in range(RANK):
        cps.append(pltpu.make_async_copy(
            t_ref.at[f],
            flat_ref.at[pl.ds(f * FSTRIDE + c * DW, DW)],
            sem,
        ))
    for cp in cps:
        cp.start()
    for cp in cps:
        cp.wait()


def _detile(t):
    return pl.pallas_call(
        _detile_body,
        grid=(NCH,),
        in_specs=[pl.BlockSpec((RANK, DW), lambda c: (0, c))],
        out_specs=pl.BlockSpec(memory_space=pl.ANY),
        out_shape=jax.ShapeDtypeStruct((RANK * FSTRIDE,), jnp.float32),
        scratch_shapes=[pltpu.SemaphoreType.DMA],
    )(t)


def _gather_feats(rows, idx_v, buf_v, sem):
    def blk(b, carry):
        ids = idx_v[pl.ds(b * L, L)]
        d = pl.ds(b * L, L)
        for f in range(RANK):
            pltpu.async_copy(rows[f].at[ids], buf_v.at[f, d], sem)
        return carry

    lax.fori_loop(0, BLKS, blk, 0)


@functools.partial(
    pl.kernel,
    mesh=_mesh,
    out_type=(
        jax.ShapeDtypeStruct((RANK * B,), jnp.float32),
        jax.ShapeDtypeStruct((RANK * B,), jnp.float32),
    ),
    scratch_types=[
        pltpu.VMEM((BPW,), jnp.int32),
        pltpu.VMEM((BPW,), jnp.int32),
        pltpu.VMEM((RANK, BPW), jnp.float32),
        pltpu.VMEM((RANK, BPW), jnp.float32),
        pltpu.SemaphoreType.DMA,
    ],
    compiler_params=_SC_PARAMS,
)
def _sc_items(it_hbm, pids_hbm, nids_hbm, pfeat_hbm, nfeat_hbm,
              pidx_v, nidx_v, pbuf_v, nbuf_v, sem):
    wid = lax.axis_index("s") * NC + lax.axis_index("c")
    sl = pl.ds(wid * BPW, BPW)

    pltpu.sync_copy(pids_hbm.at[sl], pidx_v)
    pltpu.sync_copy(nids_hbm.at[sl], nidx_v)

    irows = [it_hbm.at[pl.ds(f * FSTRIDE, NROWS)] for f in range(RANK)]
    _gather_feats(irows, pidx_v, pbuf_v, sem)
    _gather_feats(irows, nidx_v, nbuf_v, sem)

    proto = it_hbm.at[pl.ds(0, RANK * BPW)]
    pltpu.make_async_copy(proto, pbuf_v, sem).wait()
    pltpu.make_async_copy(proto, nbuf_v, sem).wait()

    for f in range(RANK):
        out = pl.ds(f * B + wid * BPW, BPW)
        pltpu.sync_copy(pbuf_v.at[f], pfeat_hbm.at[out])
        pltpu.sync_copy(nbuf_v.at[f], nfeat_hbm.at[out])


@functools.partial(
    pl.kernel,
    mesh=_mesh,
    out_type=(
        jax.ShapeDtypeStruct((B,), jnp.float32),
        jax.ShapeDtypeStruct((B,), jnp.float32),
    ),
    scratch_types=[
        pltpu.VMEM((BPW,), jnp.int32),
        pltpu.VMEM((RANK, BPW), jnp.float32),
        pltpu.VMEM((RANK, BPW), jnp.float32),
        pltpu.VMEM((RANK, BPW), jnp.float32),
        pltpu.VMEM((BPW,), jnp.float32),
        pltpu.VMEM((BPW,), jnp.float32),
        pltpu.SemaphoreType.DMA,
        pltpu.SemaphoreType.DMA,
    ],
    compiler_params=_SC_PARAMS,
)
def _sc_user_score(ut_hbm, uids_hbm, pfeat_hbm, nfeat_hbm,
                   outp_hbm, outn_hbm,
                   uidx_v, ubuf_v, pbuf_v, nbuf_v,
                   outp_v, outn_v, semG, semS):
    wid = lax.axis_index("s") * NC + lax.axis_index("c")
    sl = pl.ds(wid * BPW, BPW)

    pltpu.sync_copy(uids_hbm.at[sl], uidx_v)

    urows = [ut_hbm.at[pl.ds(f * FSTRIDE, NROWS)] for f in range(RANK)]
    _gather_feats(urows, uidx_v, ubuf_v, semG)

    stage = []
    for f in range(RANK):
        src = pl.ds(f * B + wid * BPW, BPW)
        stage.append(pltpu.async_copy(pfeat_hbm.at[src], pbuf_v.at[f], semS))
        stage.append(pltpu.async_copy(nfeat_hbm.at[src], nbuf_v.at[f], semS))
    for c in stage:
        c.wait()
    proto = ut_hbm.at[pl.ds(0, RANK * BPW)]
    pltpu.make_async_copy(proto, ubuf_v, semG).wait()

    def blk_body(b, carry):
        base = b * L
        accp = jnp.zeros((L,), jnp.float32)
        accn = jnp.zeros((L,), jnp.float32)
        for f in range(RANK):
            u = ubuf_v[f, pl.ds(base, L)]
            p = pbuf_v[f, pl.ds(base, L)]
            n = nbuf_v[f, pl.ds(base, L)]
            accp = accp + u * p
            accn = accn + u * n
        outp_v[pl.ds(base, L)] = accp
        outn_v[pl.ds(base, L)] = accn
        return carry

    lax.fori_loop(0, BLKS, blk_body, 0)

    pltpu.sync_copy(outp_v, outp_hbm.at[sl])
    pltpu.sync_copy(outn_v, outn_hbm.at[sl])


def kernel(user_ids, pos_items, neg_items, user_emb, item_emb):
    iflat = _detile(item_emb.T)
    pfeat, nfeat = _sc_items(iflat,
                             pos_items.astype(jnp.int32),
                             neg_items.astype(jnp.int32))
    uflat = _detile(user_emb.T)
    return _sc_user_score(uflat, user_ids.astype(jnp.int32), pfeat, nfeat)
